# baseline (speedup 1.0000x reference)
import functools

import jax
import jax.numpy as jnp
from jax import lax
from jax.experimental import pallas as pl
from jax.experimental.pallas import tpu as pltpu
from jax.experimental.pallas import tpu_sc as plsc

B, MB, ML = 1024, 64, 64
V, DC, DM = 100000, 64, 256
NBARS = B * MB
L = 16
NLC = DC // L

NC, NS = 2, 16
HALF_V = V // 2
SHARD = 50048
STAGE = SHARD // NS

BARS_PER_TILE = NBARS // NS
CB = 64
CHUNKS = BARS_PER_TILE // CB
GROUP = 2
NGRP = CB // GROUP

_PERM = ([2 * i for i in range(16)] + [2 * i + 1 for i in range(16)]
         + [32 + 2 * i for i in range(16)] + [33 + 2 * i for i in range(16)])


def _sc_pool_body(idx_hbm, msk_hbm, tbl_hbm, pooled_hbm,
                  idx_v, msk_v, idxe_v, rows0_v, rows1_v, rows2_v, rows3_v,
                  outc_v, spm_v, sem0, sem1, sem2, sem3):
    c = lax.axis_index("c")
    sid = lax.axis_index("s")
    pltpu.sync_copy(tbl_hbm.at[pl.ds(c * SHARD + sid * STAGE, STAGE)],
                    spm_v.at[pl.ds(sid * STAGE, STAGE)])
    plsc.subcore_barrier()

    vbase = c * HALF_V
    tile_bar0 = sid * BARS_PER_TILE
    rows_ring = (rows0_v, rows1_v, rows2_v, rows3_v)
    sem_ring = (sem0, sem1, sem2, sem3)
    NSLOT = len(rows_ring)

    def prep(g, s):
        for b in range(GROUP):
            for q in range(NLC):
                iv = idx_v[g * GROUP + b, pl.ds(q * L, L)]
                m = msk_v[g * GROUP + b, pl.ds(q * L, L)]
                lo = iv - vbase
                valid = (lo >= 0) & (lo < HALF_V) & (m > 0)
                idxe_v[s, pl.ds(b * ML + q * L, L)] = jnp.where(
                    valid, lo, HALF_V)

    def fire(s):
        pltpu.async_copy(spm_v.at[idxe_v.at[s]], rows_ring[s], sem_ring[s])

    def gwait(s):
        pltpu.make_async_copy(spm_v.at[idxe_v.at[s]], rows_ring[s],
                              sem_ring[s]).wait()

    def accum(g, s):
        rows = rows_ring[s]
        for b in range(GROUP):
            acc = [jnp.zeros((L,), jnp.float32) for _ in range(NLC)]
            for j in range(0, ML, 2):
                for h in range(2):
                    rv = (rows[b * ML + j, pl.ds(32 * h, 32)]
                          + rows[b * ML + j + 1, pl.ds(32 * h, 32)])
                    ua, ub = plsc.unpack(rv, format=plsc.PackFormat.INTERLEAVED)
                    acc[2 * h] = acc[2 * h] + ua
                    acc[2 * h + 1] = acc[2 * h + 1] + ub
            for k in range(NLC):
                outc_v[g * GROUP + b, pl.ds(k * L, L)] = acc[k]

    def chunk_body(ci, _):
        bar0 = tile_bar0 + ci * CB
        pltpu.sync_copy(idx_hbm.at[pl.ds(bar0, CB)], idx_v)
        pltpu.sync_copy(msk_hbm.at[pl.ds(bar0, CB)], msk_v)
        for k in range(NSLOT - 1):
            prep(k, k)
            fire(k)

        def ring_body(h, _):
            for k in range(NSLOT):
                g = NSLOT * h + k

                @pl.when(g + NSLOT - 1 < NGRP)
                def _():
                    prep(g + NSLOT - 1, (k + NSLOT - 1) % NSLOT)
                    fire((k + NSLOT - 1) % NSLOT)

                gwait(k)
                accum(g, k)
            return 0

        lax.fori_loop(0, NGRP // NSLOT, ring_body, 0)
        pltpu.sync_copy(outc_v, pooled_hbm.at[c, pl.ds(bar0, CB)])
        return 0

    lax.fori_loop(0, CHUNKS, chunk_body, 0)


_sc_pool = functools.partial(
    pl.kernel,
    out_type=jax.ShapeDtypeStruct((NC, NBARS, DC), jnp.float32),
    mesh=plsc.VectorSubcoreMesh(core_axis_name="c", subcore_axis_name="s"),
    scratch_types=[
        pltpu.VMEM((CB, ML), jnp.int32),
        pltpu.VMEM((CB, ML), jnp.int32),
        pltpu.VMEM((4, GROUP * ML), jnp.int32),
        pltpu.VMEM((GROUP * ML, DC), jnp.bfloat16),
        pltpu.VMEM((GROUP * ML, DC), jnp.bfloat16),
        pltpu.VMEM((GROUP * ML, DC), jnp.bfloat16),
        pltpu.VMEM((GROUP * ML, DC), jnp.bfloat16),
        pltpu.VMEM((CB, DC), jnp.float32),
        pltpu.VMEM_SHARED((SHARD, DC), jnp.bfloat16),
        pltpu.SemaphoreType.DMA,
        pltpu.SemaphoreType.DMA,
        pltpu.SemaphoreType.DMA,
        pltpu.SemaphoreType.DMA,
    ],
    compiler_params=pltpu.CompilerParams(use_tc_tiling_on_sc=False,
                                         needs_layout_passes=False),
)(_sc_pool_body)


def _tc_body(x0_ref, x1_ref, m_ref, w_ref, b_ref, pos_ref, g_ref, bb_ref,
             o_ref):
    x = x0_ref[0] + x1_ref[0]
    cnt = jnp.sum(m_ref[...].astype(jnp.float32), axis=-1, keepdims=True)
    rinv = 1.0 / jnp.maximum(cnt, 1.0)
    y = jnp.dot(x, w_ref[...], preferred_element_type=jnp.float32)
    y = y * rinv + b_ref[...]
    r = x.shape[0] // MB
    y = (y.reshape(r, MB, DM) + pos_ref[...][None]).reshape(r * MB, DM)
    mu = jnp.mean(y, axis=-1, keepdims=True)
    d = y - mu
    var = jnp.mean(d * d, axis=-1, keepdims=True)
    o_ref[...] = d * lax.rsqrt(var + 1e-5) * g_ref[...] + bb_ref[...]


def kernel(bar_indices, char_mask, bar_mask, char_table, W, b, pos_table,
           gamma, beta):
    idx = bar_indices.astype(jnp.int32).reshape(NBARS, ML)
    msk = char_mask.astype(jnp.int32).reshape(NBARS, ML)

    tb = char_table.astype(jnp.bfloat16)
    zpad = jnp.zeros((SHARD - HALF_V, DC), jnp.bfloat16)
    tbl2 = jnp.concatenate([tb[:HALF_V], zpad, tb[HALF_V:], zpad], axis=0)

    pooled = _sc_pool(idx, msk, tbl2)

    w_perm = W[jnp.asarray(_PERM), :]

    R = 512
    out = pl.pallas_call(
        _tc_body,
        grid=(NBARS // R,),
        in_specs=[
            pl.BlockSpec((1, R, DC), lambda i: (0, i, 0)),
            pl.BlockSpec((1, R, DC), lambda i: (1, i, 0)),
            pl.BlockSpec((R, ML), lambda i: (i, 0)),
            pl.BlockSpec((DC, DM), lambda i: (0, 0)),
            pl.BlockSpec((1, DM), lambda i: (0, 0)),
            pl.BlockSpec((MB, DM), lambda i: (0, 0)),
            pl.BlockSpec((1, DM), lambda i: (0, 0)),
            pl.BlockSpec((1, DM), lambda i: (0, 0)),
        ],
        out_specs=pl.BlockSpec((R, DM), lambda i: (i, 0)),
        out_shape=jax.ShapeDtypeStruct((NBARS, DM), jnp.float32),
    )(pooled, pooled, msk, w_perm, b.reshape(1, DM), pos_table[:MB],
      gamma.reshape(1, DM), beta.reshape(1, DM))

    return out.reshape(B, MB, DM), bar_mask


# speedup vs baseline: 1.0043x; 1.0043x over previous
import functools

import jax
import jax.numpy as jnp
from jax import lax
from jax.experimental import pallas as pl
from jax.experimental.pallas import tpu as pltpu
from jax.experimental.pallas import tpu_sc as plsc

B, MB, ML = 1024, 64, 64
V, DC, DM = 100000, 64, 256
NBARS = B * MB
L = 16
NLC = DC // L

NC, NS = 2, 16
HALF_V = V // 2
SHARD = 50048
STAGE = SHARD // NS

BARS_PER_TILE = NBARS // NS
CB = 32
CHUNKS = BARS_PER_TILE // CB
GROUP = 2
NGRP = CB // GROUP
NSLOT = 4

_PERM = ([2 * i for i in range(16)] + [2 * i + 1 for i in range(16)]
         + [32 + 2 * i for i in range(16)] + [33 + 2 * i for i in range(16)])


def _sc_pool_body(idx_hbm, msk_hbm, tbl_hbm, pooled_hbm,
                  idxA, idxB, mskA, mskB, idxe_v,
                  rows0_v, rows1_v, rows2_v, rows3_v, outA, outB, spm_v,
                  gs0, gs1, gs2, gs3, stgA, stgB, owA, owB):
    c = lax.axis_index("c")
    sid = lax.axis_index("s")
    pltpu.sync_copy(tbl_hbm.at[pl.ds(c * SHARD + sid * STAGE, STAGE)],
                    spm_v.at[pl.ds(sid * STAGE, STAGE)])
    plsc.subcore_barrier()

    vbase = c * HALF_V
    tile_bar0 = sid * BARS_PER_TILE
    rows_ring = (rows0_v, rows1_v, rows2_v, rows3_v)
    gsem = (gs0, gs1, gs2, gs3)
    idx_bufs, msk_bufs = (idxA, idxB), (mskA, mskB)
    out_bufs, stg_sems, ow_sems = (outA, outB), (stgA, stgB), (owA, owB)

    def prep(idxb, mskb, g, s):
        for b in range(GROUP):
            for q in range(NLC):
                iv = idxb[g * GROUP + b, pl.ds(q * L, L)]
                m = mskb[g * GROUP + b, pl.ds(q * L, L)]
                lo = iv - vbase
                valid = (lo >= 0) & (lo < HALF_V) & (m > 0)
                idxe_v[s, pl.ds(b * ML + q * L, L)] = jnp.where(
                    valid, lo, HALF_V)

    def fire(s):
        pltpu.async_copy(spm_v.at[idxe_v.at[s]], rows_ring[s], gsem[s])

    def gwait(s):
        pltpu.make_async_copy(spm_v.at[idxe_v.at[s]], rows_ring[s],
                              gsem[s]).wait()

    def accum(outb, g, s):
        rows = rows_ring[s]
        for b in range(GROUP):
            def jbody(jj, acc):
                base = b * ML + jj * 8
                for pr in range(4):
                    for h in range(2):
                        rv = (rows[base + 2 * pr, pl.ds(32 * h, 32)]
                              + rows[base + 2 * pr + 1, pl.ds(32 * h, 32)])
                        ua, ub = plsc.unpack(
                            rv, format=plsc.PackFormat.INTERLEAVED)
                        acc = (acc[:2 * h] + (acc[2 * h] + ua, acc[2 * h + 1] + ub)
                               + acc[2 * h + 2:])
                return acc
            z = jnp.zeros((L,), jnp.float32)
            acc = lax.fori_loop(0, ML // 8, jbody, (z, z, z, z))
            for k in range(NLC):
                outb[g * GROUP + b, pl.ds(k * L, L)] = acc[k]

    def stage_fire(ci, p):
        bar0 = tile_bar0 + ci * CB
        pltpu.async_copy(idx_hbm.at[pl.ds(bar0, CB)], idx_bufs[p],
                         stg_sems[p])
        pltpu.async_copy(msk_hbm.at[pl.ds(bar0, CB)], msk_bufs[p],
                         stg_sems[p])

    def stage_wait(p):
        pltpu.make_async_copy(idx_hbm.at[pl.ds(tile_bar0, CB)], idx_bufs[p],
                              stg_sems[p]).wait()
        pltpu.make_async_copy(msk_hbm.at[pl.ds(tile_bar0, CB)], msk_bufs[p],
                              stg_sems[p]).wait()

    def out_fire(ci, p):
        pltpu.async_copy(out_bufs[p],
                         pooled_hbm.at[c, pl.ds(tile_bar0 + ci * CB, CB)],
                         ow_sems[p])

    def out_wait(p):
        pltpu.make_async_copy(out_bufs[p],
                              pooled_hbm.at[c, pl.ds(tile_bar0, CB)],
                              ow_sems[p]).wait()

    stage_fire(0, 0)
    stage_wait(0)
    for k in range(NSLOT):
        prep(idxA, mskA, k, k)
        fire(k)

    def chunk_pair(t, _):
        for p in range(2):
            ci = 2 * t + p
            cur_idx, cur_msk, cur_out = idx_bufs[p], msk_bufs[p], out_bufs[p]

            @pl.when(ci + 1 < CHUNKS)
            def _():
                stage_fire(ci + 1, 1 - p)

            @pl.when(ci >= 2)
            def _():
                out_wait(p)

            def main_body(h, _):
                for k in range(NSLOT):
                    g = NSLOT * h + k
                    gwait(k)
                    accum(cur_out, g, k)
                    prep(cur_idx, cur_msk, g + NSLOT, k)
                    fire(k)
                return 0

            lax.fori_loop(0, NGRP // NSLOT - 1, main_body, 0)

            @pl.when(ci + 1 < CHUNKS)
            def _():
                stage_wait(1 - p)

            for k in range(NSLOT):
                gwait(k)
                accum(cur_out, NGRP - NSLOT + k, k)

                @pl.when(ci + 1 < CHUNKS)
                def _():
                    prep(idx_bufs[1 - p], msk_bufs[1 - p], k, k)
                    fire(k)

            out_fire(ci, p)
        return 0

    lax.fori_loop(0, CHUNKS // 2, chunk_pair, 0)
    out_wait(0)
    out_wait(1)


_sc_pool = functools.partial(
    pl.kernel,
    out_type=jax.ShapeDtypeStruct((NC, NBARS, DC), jnp.float32),
    mesh=plsc.VectorSubcoreMesh(core_axis_name="c", subcore_axis_name="s"),
    scratch_types=[
        pltpu.VMEM((CB, ML), jnp.int32),
        pltpu.VMEM((CB, ML), jnp.int32),
        pltpu.VMEM((CB, ML), jnp.int32),
        pltpu.VMEM((CB, ML), jnp.int32),
        pltpu.VMEM((NSLOT, GROUP * ML), jnp.int32),
        pltpu.VMEM((GROUP * ML, DC), jnp.bfloat16),
        pltpu.VMEM((GROUP * ML, DC), jnp.bfloat16),
        pltpu.VMEM((GROUP * ML, DC), jnp.bfloat16),
        pltpu.VMEM((GROUP * ML, DC), jnp.bfloat16),
        pltpu.VMEM((CB, DC), jnp.float32),
        pltpu.VMEM((CB, DC), jnp.float32),
        pltpu.VMEM_SHARED((SHARD, DC), jnp.bfloat16),
        pltpu.SemaphoreType.DMA,
        pltpu.SemaphoreType.DMA,
        pltpu.SemaphoreType.DMA,
        pltpu.SemaphoreType.DMA,
        pltpu.SemaphoreType.DMA,
        pltpu.SemaphoreType.DMA,
        pltpu.SemaphoreType.DMA,
        pltpu.SemaphoreType.DMA,
    ],
    compiler_params=pltpu.CompilerParams(use_tc_tiling_on_sc=False,
                                         needs_layout_passes=False),
)(_sc_pool_body)


def _tc_body(x0_ref, x1_ref, m_ref, w_ref, b_ref, pos_ref, g_ref, bb_ref,
             o_ref):
    x = x0_ref[0] + x1_ref[0]
    cnt = jnp.sum(m_ref[...].astype(jnp.float32), axis=-1, keepdims=True)
    rinv = 1.0 / jnp.maximum(cnt, 1.0)
    y = jnp.dot(x, w_ref[...], preferred_element_type=jnp.float32)
    y = y * rinv + b_ref[...]
    r = x.shape[0] // MB
    y = (y.reshape(r, MB, DM) + pos_ref[...][None]).reshape(r * MB, DM)
    mu = jnp.mean(y, axis=-1, keepdims=True)
    d = y - mu
    var = jnp.mean(d * d, axis=-1, keepdims=True)
    o_ref[...] = d * lax.rsqrt(var + 1e-5) * g_ref[...] + bb_ref[...]


def kernel(bar_indices, char_mask, bar_mask, char_table, W, b, pos_table,
           gamma, beta):
    idx = bar_indices.astype(jnp.int32).reshape(NBARS, ML)
    msk = char_mask.astype(jnp.int32).reshape(NBARS, ML)

    tb = char_table.astype(jnp.bfloat16)
    zpad = jnp.zeros((SHARD - HALF_V, DC), jnp.bfloat16)
    tbl2 = jnp.concatenate([tb[:HALF_V], zpad, tb[HALF_V:], zpad], axis=0)

    pooled = _sc_pool(idx, msk, tbl2)

    w_perm = W[jnp.asarray(_PERM), :]

    R = 512
    out = pl.pallas_call(
        _tc_body,
        grid=(NBARS // R,),
        in_specs=[
            pl.BlockSpec((1, R, DC), lambda i: (0, i, 0)),
            pl.BlockSpec((1, R, DC), lambda i: (1, i, 0)),
            pl.BlockSpec((R, ML), lambda i: (i, 0)),
            pl.BlockSpec((DC, DM), lambda i: (0, 0)),
            pl.BlockSpec((1, DM), lambda i: (0, 0)),
            pl.BlockSpec((MB, DM), lambda i: (0, 0)),
            pl.BlockSpec((1, DM), lambda i: (0, 0)),
            pl.BlockSpec((1, DM), lambda i: (0, 0)),
        ],
        out_specs=pl.BlockSpec((R, DM), lambda i: (i, 0)),
        out_shape=jax.ShapeDtypeStruct((NBARS, DM), jnp.float32),
    )(pooled, pooled, msk, w_perm, b.reshape(1, DM), pos_table[:MB],
      gamma.reshape(1, DM), beta.reshape(1, DM))

    return out.reshape(B, MB, DM), bar_mask


# speedup vs baseline: 2.1260x; 2.1170x over previous
import functools

import jax
import jax.numpy as jnp
from jax import lax
from jax.experimental import pallas as pl
from jax.experimental.pallas import tpu as pltpu
from jax.experimental.pallas import tpu_sc as plsc

B, MB, ML = 1024, 64, 64
V, DC, DM = 100000, 64, 256
NBARS = B * MB
L = 16
NLC = DC // L

NC, NS = 2, 16
HALF_V = V // 2
SHARD = 50048
STAGE = SHARD // NS

BARS_PER_TILE = NBARS // NS
CB = 32
CHUNKS = BARS_PER_TILE // CB
NSLOT = 4
SUB = 32

_PERM = ([2 * i for i in range(16)] + [2 * i + 1 for i in range(16)]
         + [32 + 2 * i for i in range(16)] + [33 + 2 * i for i in range(16)])


def _sc_pool_body(idx_hbm, msk_hbm, tbl_hbm, pooled_hbm,
                  idxA, idxB, mskA, mskB, idxe_v,
                  rows0_v, rows1_v, rows2_v, rows3_v, outA, outB, spm_v,
                  cnt_sm, gs0, gs1, gs2, gs3, stgA, stgB, owA, owB):
    c = lax.axis_index("c")
    sid = lax.axis_index("s")
    pltpu.sync_copy(tbl_hbm.at[pl.ds(c * SHARD + sid * STAGE, STAGE)],
                    spm_v.at[pl.ds(sid * STAGE, STAGE)])
    plsc.subcore_barrier()

    vbase = c * HALF_V
    tile_bar0 = sid * BARS_PER_TILE
    rows_ring = (rows0_v, rows1_v, rows2_v, rows3_v)
    gsem = (gs0, gs1, gs2, gs3)
    idx_bufs, msk_bufs = (idxA, idxB), (mskA, mskB)
    out_bufs, stg_sems, ow_sems = (outA, outB), (stgA, stgB), (owA, owB)

    dummy = jnp.full((L,), HALF_V, jnp.int32)

    def prep(idxb, mskb, g, s):
        for q in range(NLC):
            idxe_v[s, pl.ds(q * L, L)] = dummy
        n = jnp.int32(0)
        for q in range(NLC):
            iv = idxb[g, pl.ds(q * L, L)]
            m = mskb[g, pl.ds(q * L, L)]
            lo = iv - vbase
            valid = (lo >= 0) & (lo < HALF_V) & (m > 0)
            plsc.store_compressed(idxe_v.at[s, pl.ds(n, L)], lo, mask=valid)
            n = n + plsc.all_reduce_population_count(valid)[0]
        cnt_sm[s, 0] = n

        @pl.when(n > 0)
        def _():
            pltpu.async_copy(spm_v.at[idxe_v.at[s, pl.ds(0, SUB)]],
                             rows_ring[s].at[pl.ds(0, SUB)], gsem[s])

        @pl.when(n > SUB)
        def _():
            pltpu.async_copy(spm_v.at[idxe_v.at[s, pl.ds(SUB, SUB)]],
                             rows_ring[s].at[pl.ds(SUB, SUB)], gsem[s])

    def gwait(s):
        n = cnt_sm[s, 0]

        @pl.when(n > 0)
        def _():
            pltpu.make_async_copy(spm_v.at[idxe_v.at[s, pl.ds(0, SUB)]],
                                  rows_ring[s].at[pl.ds(0, SUB)],
                                  gsem[s]).wait()

        @pl.when(n > SUB)
        def _():
            pltpu.make_async_copy(spm_v.at[idxe_v.at[s, pl.ds(0, SUB)]],
                                  rows_ring[s].at[pl.ds(0, SUB)],
                                  gsem[s]).wait()

    def accum(outb, g, s):
        rows = rows_ring[s]
        npair = (cnt_sm[s, 0] + 1) >> 1

        def jbody(jj, acc):
            for h in range(2):
                rv = (rows[2 * jj, pl.ds(32 * h, 32)]
                      + rows[2 * jj + 1, pl.ds(32 * h, 32)])
                ua, ub = plsc.unpack(rv, format=plsc.PackFormat.INTERLEAVED)
                acc = (acc[:2 * h] + (acc[2 * h] + ua, acc[2 * h + 1] + ub)
                       + acc[2 * h + 2:])
            return acc

        z = jnp.zeros((L,), jnp.float32)
        acc = lax.fori_loop(0, npair, jbody, (z, z, z, z))
        for k in range(NLC):
            outb[g, pl.ds(k * L, L)] = acc[k]

    def stage_fire(ci, p):
        bar0 = tile_bar0 + ci * CB
        pltpu.async_copy(idx_hbm.at[pl.ds(bar0, CB)], idx_bufs[p],
                         stg_sems[p])
        pltpu.async_copy(msk_hbm.at[pl.ds(bar0, CB)], msk_bufs[p],
                         stg_sems[p])

    def stage_wait(p):
        pltpu.make_async_copy(idx_hbm.at[pl.ds(tile_bar0, CB)], idx_bufs[p],
                              stg_sems[p]).wait()
        pltpu.make_async_copy(msk_hbm.at[pl.ds(tile_bar0, CB)], msk_bufs[p],
                              stg_sems[p]).wait()

    def out_fire(ci, p):
        pltpu.async_copy(out_bufs[p],
                         pooled_hbm.at[c, pl.ds(tile_bar0 + ci * CB, CB)],
                         ow_sems[p])

    def out_wait(p):
        pltpu.make_async_copy(out_bufs[p],
                              pooled_hbm.at[c, pl.ds(tile_bar0, CB)],
                              ow_sems[p]).wait()

    stage_fire(0, 0)
    stage_wait(0)
    for k in range(NSLOT):
        prep(idxA, mskA, k, k)

    def chunk_pair(t, _):
        for p in range(2):
            ci = 2 * t + p
            cur_idx, cur_msk, cur_out = idx_bufs[p], msk_bufs[p], out_bufs[p]

            @pl.when(ci + 1 < CHUNKS)
            def _():
                stage_fire(ci + 1, 1 - p)

            @pl.when(ci >= 2)
            def _():
                out_wait(p)

            def main_body(h, _):
                for k in range(NSLOT):
                    g = NSLOT * h + k
                    gwait(k)
                    accum(cur_out, g, k)
                    prep(cur_idx, cur_msk, g + NSLOT, k)
                return 0

            lax.fori_loop(0, CB // NSLOT - 1, main_body, 0)

            @pl.when(ci + 1 < CHUNKS)
            def _():
                stage_wait(1 - p)

            for k in range(NSLOT):
                gwait(k)
                accum(cur_out, CB - NSLOT + k, k)

                @pl.when(ci + 1 < CHUNKS)
                def _():
                    prep(idx_bufs[1 - p], msk_bufs[1 - p], k, k)

            out_fire(ci, p)
        return 0

    lax.fori_loop(0, CHUNKS // 2, chunk_pair, 0)
    out_wait(0)
    out_wait(1)


_sc_pool = functools.partial(
    pl.kernel,
    out_type=jax.ShapeDtypeStruct((NC, NBARS, DC), jnp.float32),
    mesh=plsc.VectorSubcoreMesh(core_axis_name="c", subcore_axis_name="s"),
    scratch_types=[
        pltpu.VMEM((CB, ML), jnp.int32),
        pltpu.VMEM((CB, ML), jnp.int32),
        pltpu.VMEM((CB, ML), jnp.int32),
        pltpu.VMEM((CB, ML), jnp.int32),
        pltpu.VMEM((NSLOT, ML), jnp.int32),
        pltpu.VMEM((ML, DC), jnp.bfloat16),
        pltpu.VMEM((ML, DC), jnp.bfloat16),
        pltpu.VMEM((ML, DC), jnp.bfloat16),
        pltpu.VMEM((ML, DC), jnp.bfloat16),
        pltpu.VMEM((CB, DC), jnp.float32),
        pltpu.VMEM((CB, DC), jnp.float32),
        pltpu.VMEM_SHARED((SHARD, DC), jnp.bfloat16),
        pltpu.SMEM((NSLOT, 1), jnp.int32),
        pltpu.SemaphoreType.DMA,
        pltpu.SemaphoreType.DMA,
        pltpu.SemaphoreType.DMA,
        pltpu.SemaphoreType.DMA,
        pltpu.SemaphoreType.DMA,
        pltpu.SemaphoreType.DMA,
        pltpu.SemaphoreType.DMA,
        pltpu.SemaphoreType.DMA,
    ],
    compiler_params=pltpu.CompilerParams(use_tc_tiling_on_sc=False,
                                         needs_layout_passes=False),
)(_sc_pool_body)


def _tc_body(x0_ref, x1_ref, m_ref, w_ref, b_ref, pos_ref, g_ref, bb_ref,
             o_ref):
    x = x0_ref[0] + x1_ref[0]
    cnt = jnp.sum(m_ref[...].astype(jnp.float32), axis=-1, keepdims=True)
    rinv = 1.0 / jnp.maximum(cnt, 1.0)
    y = jnp.dot(x, w_ref[...], preferred_element_type=jnp.float32)
    y = y * rinv + b_ref[...]
    r = x.shape[0] // MB
    y = (y.reshape(r, MB, DM) + pos_ref[...][None]).reshape(r * MB, DM)
    mu = jnp.mean(y, axis=-1, keepdims=True)
    d = y - mu
    var = jnp.mean(d * d, axis=-1, keepdims=True)
    o_ref[...] = d * lax.rsqrt(var + 1e-5) * g_ref[...] + bb_ref[...]


def kernel(bar_indices, char_mask, bar_mask, char_table, W, b, pos_table,
           gamma, beta):
    idx = bar_indices.astype(jnp.int32).reshape(NBARS, ML)
    msk = char_mask.astype(jnp.int32).reshape(NBARS, ML)

    tb = char_table.astype(jnp.bfloat16)
    zpad = jnp.zeros((SHARD - HALF_V, DC), jnp.bfloat16)
    tbl2 = jnp.concatenate([tb[:HALF_V], zpad, tb[HALF_V:], zpad], axis=0)

    pooled = _sc_pool(idx, msk, tbl2)

    w_perm = W[jnp.asarray(_PERM), :]

    R = 512
    out = pl.pallas_call(
        _tc_body,
        grid=(NBARS // R,),
        in_specs=[
            pl.BlockSpec((1, R, DC), lambda i: (0, i, 0)),
            pl.BlockSpec((1, R, DC), lambda i: (1, i, 0)),
            pl.BlockSpec((R, ML), lambda i: (i, 0)),
            pl.BlockSpec((DC, DM), lambda i: (0, 0)),
            pl.BlockSpec((1, DM), lambda i: (0, 0)),
            pl.BlockSpec((MB, DM), lambda i: (0, 0)),
            pl.BlockSpec((1, DM), lambda i: (0, 0)),
            pl.BlockSpec((1, DM), lambda i: (0, 0)),
        ],
        out_specs=pl.BlockSpec((R, DM), lambda i: (i, 0)),
        out_shape=jax.ShapeDtypeStruct((NBARS, DM), jnp.float32),
    )(pooled, pooled, msk, w_perm, b.reshape(1, DM), pos_table[:MB],
      gamma.reshape(1, DM), beta.reshape(1, DM))

    return out.reshape(B, MB, DM), bar_mask


# speedup vs baseline: 2.1268x; 1.0004x over previous
import functools

import jax
import jax.numpy as jnp
from jax import lax
from jax.experimental import pallas as pl
from jax.experimental.pallas import tpu as pltpu
from jax.experimental.pallas import tpu_sc as plsc

B, MB, ML = 1024, 64, 64
V, DC, DM = 100000, 64, 256
NBARS = B * MB
L = 16
NLC = DC // L

NC, NS = 2, 16
HALF_V = V // 2
SHARD = 50048
STAGE = SHARD // NS

BARS_PER_TILE = NBARS // NS
CB = 32
CHUNKS = BARS_PER_TILE // CB
NSLOT = 4
SUB = 32

_PERM = ([2 * i for i in range(16)] + [2 * i + 1 for i in range(16)]
         + [32 + 2 * i for i in range(16)] + [33 + 2 * i for i in range(16)])


def _sc_pool_body(idx_hbm, tbl_hbm, pooled_hbm,
                  idxA, idxB, idxe_v,
                  rows0_v, rows1_v, rows2_v, rows3_v, outA, outB, spm_v,
                  cnt_sm, gs0, gs1, gs2, gs3, stgA, stgB, owA, owB):
    c = lax.axis_index("c")
    sid = lax.axis_index("s")
    pltpu.sync_copy(tbl_hbm.at[pl.ds(c * SHARD + sid * STAGE, STAGE)],
                    spm_v.at[pl.ds(sid * STAGE, STAGE)])
    plsc.subcore_barrier()

    vbase = c * HALF_V
    tile_bar0 = sid * BARS_PER_TILE
    rows_ring = (rows0_v, rows1_v, rows2_v, rows3_v)
    gsem = (gs0, gs1, gs2, gs3)
    idx_bufs = (idxA, idxB)
    out_bufs, stg_sems, ow_sems = (outA, outB), (stgA, stgB), (owA, owB)

    dummy = jnp.full((L,), HALF_V, jnp.int32)

    def prep(idxb, g, s):
        for q in range(NLC):
            idxe_v[s, pl.ds(q * L, L)] = dummy
        n = jnp.int32(0)
        for q in range(NLC):
            iv = idxb[g, pl.ds(q * L, L)]
            lo = iv - vbase
            valid = (lo >= 0) & (lo < HALF_V)
            plsc.store_compressed(idxe_v.at[s, pl.ds(n, L)], lo, mask=valid)
            n = n + plsc.all_reduce_population_count(valid)[0]
        cnt_sm[s, 0] = n

        @pl.when(n > 0)
        def _():
            pltpu.async_copy(spm_v.at[idxe_v.at[s, pl.ds(0, SUB)]],
                             rows_ring[s].at[pl.ds(0, SUB)], gsem[s])

        @pl.when(n > SUB)
        def _():
            pltpu.async_copy(spm_v.at[idxe_v.at[s, pl.ds(SUB, SUB)]],
                             rows_ring[s].at[pl.ds(SUB, SUB)], gsem[s])

    def gwait(s):
        n = cnt_sm[s, 0]

        @pl.when(n > 0)
        def _():
            pltpu.make_async_copy(spm_v.at[idxe_v.at[s, pl.ds(0, SUB)]],
                                  rows_ring[s].at[pl.ds(0, SUB)],
                                  gsem[s]).wait()

        @pl.when(n > SUB)
        def _():
            pltpu.make_async_copy(spm_v.at[idxe_v.at[s, pl.ds(0, SUB)]],
                                  rows_ring[s].at[pl.ds(0, SUB)],
                                  gsem[s]).wait()

    def accum(outb, g, s):
        rows = rows_ring[s]
        npair = (cnt_sm[s, 0] + 1) >> 1

        def jbody(jj, acc):
            for h in range(2):
                rv = (rows[2 * jj, pl.ds(32 * h, 32)]
                      + rows[2 * jj + 1, pl.ds(32 * h, 32)])
                ua, ub = plsc.unpack(rv, format=plsc.PackFormat.INTERLEAVED)
                acc = (acc[:2 * h] + (acc[2 * h] + ua, acc[2 * h + 1] + ub)
                       + acc[2 * h + 2:])
            return acc

        z = jnp.zeros((L,), jnp.float32)
        acc = lax.fori_loop(0, npair, jbody, (z, z, z, z))
        for k in range(NLC):
            outb[g, pl.ds(k * L, L)] = acc[k]

    def stage_fire(ci, p):
        bar0 = tile_bar0 + ci * CB
        pltpu.async_copy(idx_hbm.at[pl.ds(bar0, CB)], idx_bufs[p],
                         stg_sems[p])

    def stage_wait(p):
        pltpu.make_async_copy(idx_hbm.at[pl.ds(tile_bar0, CB)], idx_bufs[p],
                              stg_sems[p]).wait()

    def out_fire(ci, p):
        pltpu.async_copy(out_bufs[p],
                         pooled_hbm.at[c, pl.ds(tile_bar0 + ci * CB, CB)],
                         ow_sems[p])

    def out_wait(p):
        pltpu.make_async_copy(out_bufs[p],
                              pooled_hbm.at[c, pl.ds(tile_bar0, CB)],
                              ow_sems[p]).wait()

    stage_fire(0, 0)
    stage_wait(0)
    for k in range(NSLOT):
        prep(idxA, k, k)

    def chunk_pair(t, _):
        for p in range(2):
            ci = 2 * t + p
            cur_idx, cur_out = idx_bufs[p], out_bufs[p]

            @pl.when(ci + 1 < CHUNKS)
            def _():
                stage_fire(ci + 1, 1 - p)

            @pl.when(ci >= 2)
            def _():
                out_wait(p)

            def main_body(h, _):
                for k in range(NSLOT):
                    g = NSLOT * h + k
                    gwait(k)
                    accum(cur_out, g, k)
                    prep(cur_idx, g + NSLOT, k)
                return 0

            lax.fori_loop(0, CB // NSLOT - 1, main_body, 0)

            @pl.when(ci + 1 < CHUNKS)
            def _():
                stage_wait(1 - p)

            for k in range(NSLOT):
                gwait(k)
                accum(cur_out, CB - NSLOT + k, k)

                @pl.when(ci + 1 < CHUNKS)
                def _():
                    prep(idx_bufs[1 - p], k, k)

            out_fire(ci, p)
        return 0

    lax.fori_loop(0, CHUNKS // 2, chunk_pair, 0)
    out_wait(0)
    out_wait(1)


_sc_pool = functools.partial(
    pl.kernel,
    out_type=jax.ShapeDtypeStruct((NC, NBARS, DC), jnp.float32),
    mesh=plsc.VectorSubcoreMesh(core_axis_name="c", subcore_axis_name="s"),
    scratch_types=[
        pltpu.VMEM((CB, ML), jnp.int32),
        pltpu.VMEM((CB, ML), jnp.int32),
        pltpu.VMEM((NSLOT, ML), jnp.int32),
        pltpu.VMEM((ML, DC), jnp.bfloat16),
        pltpu.VMEM((ML, DC), jnp.bfloat16),
        pltpu.VMEM((ML, DC), jnp.bfloat16),
        pltpu.VMEM((ML, DC), jnp.bfloat16),
        pltpu.VMEM((CB, DC), jnp.float32),
        pltpu.VMEM((CB, DC), jnp.float32),
        pltpu.VMEM_SHARED((SHARD, DC), jnp.bfloat16),
        pltpu.SMEM((NSLOT, 1), jnp.int32),
        pltpu.SemaphoreType.DMA,
        pltpu.SemaphoreType.DMA,
        pltpu.SemaphoreType.DMA,
        pltpu.SemaphoreType.DMA,
        pltpu.SemaphoreType.DMA,
        pltpu.SemaphoreType.DMA,
        pltpu.SemaphoreType.DMA,
        pltpu.SemaphoreType.DMA,
    ],
    compiler_params=pltpu.CompilerParams(use_tc_tiling_on_sc=False,
                                         needs_layout_passes=False),
)(_sc_pool_body)


def _tc_body(x0_ref, x1_ref, m_ref, w_ref, b_ref, pos_ref, g_ref, bb_ref,
             o_ref):
    x = x0_ref[0] + x1_ref[0]
    cnt = jnp.sum(m_ref[...].astype(jnp.float32), axis=-1, keepdims=True)
    rinv = 1.0 / jnp.maximum(cnt, 1.0)
    y = jnp.dot(x, w_ref[...], preferred_element_type=jnp.float32)
    y = y * rinv + b_ref[...]
    r = x.shape[0] // MB
    y = (y.reshape(r, MB, DM) + pos_ref[...][None]).reshape(r * MB, DM)
    mu = jnp.mean(y, axis=-1, keepdims=True)
    d = y - mu
    var = jnp.mean(d * d, axis=-1, keepdims=True)
    o_ref[...] = d * lax.rsqrt(var + 1e-5) * g_ref[...] + bb_ref[...]


def kernel(bar_indices, char_mask, bar_mask, char_table, W, b, pos_table,
           gamma, beta):
    idx = bar_indices.astype(jnp.int32).reshape(NBARS, ML)
    msk = char_mask.astype(jnp.int32).reshape(NBARS, ML)
    idxm = jnp.where(msk > 0, idx, -1)

    tb = char_table.astype(jnp.bfloat16)
    zpad = jnp.zeros((SHARD - HALF_V, DC), jnp.bfloat16)
    tbl2 = jnp.concatenate([tb[:HALF_V], zpad, tb[HALF_V:], zpad], axis=0)

    pooled = _sc_pool(idxm, tbl2)

    w_perm = W[jnp.asarray(_PERM), :]

    R = 512
    out = pl.pallas_call(
        _tc_body,
        grid=(NBARS // R,),
        in_specs=[
            pl.BlockSpec((1, R, DC), lambda i: (0, i, 0)),
            pl.BlockSpec((1, R, DC), lambda i: (1, i, 0)),
            pl.BlockSpec((R, ML), lambda i: (i, 0)),
            pl.BlockSpec((DC, DM), lambda i: (0, 0)),
            pl.BlockSpec((1, DM), lambda i: (0, 0)),
            pl.BlockSpec((MB, DM), lambda i: (0, 0)),
            pl.BlockSpec((1, DM), lambda i: (0, 0)),
            pl.BlockSpec((1, DM), lambda i: (0, 0)),
        ],
        out_specs=pl.BlockSpec((R, DM), lambda i: (i, 0)),
        out_shape=jax.ShapeDtypeStruct((NBARS, DM), jnp.float32),
    )(pooled, pooled, msk, w_perm, b.reshape(1, DM), pos_table[:MB],
      gamma.reshape(1, DM), beta.reshape(1, DM))

    return out.reshape(B, MB, DM), bar_mask
